# baseline (speedup 1.0000x reference)
import functools

import jax
import jax.numpy as jnp
from jax import lax
from jax.experimental import pallas as pl
from jax.experimental.pallas import tpu as pltpu
from jax.experimental.pallas import tpu_sc as plsc

N = 10000
NPAD = 10240
E = 320000
D = 128
DH = D // 2
K = 3
T = 4

NC, NS = 2, 16
NW = NC * NS
CH = 80
EPW = E // NW
NCH = EPW // CH
LPE = EPW // 16
RPT = NPAD // NS

_mesh = plsc.VectorSubcoreMesh(core_axis_name="c", subcore_axis_name="s")


NSEM = 5


@functools.partial(
    pl.kernel,
    out_type=jax.ShapeDtypeStruct((NC, NPAD, D), jnp.float32),
    mesh=_mesh,
    scratch_types=[
        pltpu.VMEM_SHARED((NPAD, D), jnp.float32),
        pltpu.VMEM((NCH, CH), jnp.int32),
        pltpu.VMEM((CH, D), jnp.float32),
        [pltpu.SemaphoreType.DMA for _ in range(NSEM)],
        pltpu.SemaphoreType.DMA,
    ],
)
def _deg_kernel(col3_hbm, zeros_hbm, ones_hbm, d_out, acc, colidx, ones_v,
                ssems, zsem):
    c = lax.axis_index("c")
    s = lax.axis_index("s")
    wid = c * NS + s
    pltpu.async_copy(zeros_hbm.at[pl.ds(s * RPT, RPT)],
                     acc.at[pl.ds(s * RPT, RPT)], zsem)
    pltpu.sync_copy(col3_hbm.at[wid], colidx)
    pltpu.sync_copy(ones_hbm, ones_v)
    pltpu.make_async_copy(zeros_hbm.at[pl.ds(s * RPT, RPT)],
                          acc.at[pl.ds(s * RPT, RPT)], zsem).wait()
    plsc.subcore_barrier()

    def sstart(j, i):
        pltpu.async_copy(ones_v, acc.at[colidx.at[j]], ssems[i], add=True)

    def swait(i):
        pltpu.make_async_copy(zeros_hbm.at[pl.ds(0, CH)], ones_v,
                              ssems[i]).wait()

    for i in range(NSEM):
        sstart(i, i)

    def body(m, carry):
        for i in range(NSEM):
            swait(i)
            sstart(m * NSEM + i, i)
        return carry

    lax.fori_loop(1, NCH // NSEM, body, 0)
    for i in range(NSEM):
        swait(i)
    plsc.subcore_barrier()
    pltpu.sync_copy(acc.at[pl.ds(s * RPT, RPT)], d_out.at[c, pl.ds(s * RPT, RPT)])


@functools.partial(
    pl.kernel,
    out_type=jax.ShapeDtypeStruct((NC, NPAD, D), jnp.float32),
    mesh=_mesh,
    scratch_types=[
        pltpu.VMEM_SHARED((NPAD, D), jnp.float32),
        pltpu.VMEM((EPW,), jnp.int32),
        pltpu.VMEM((NCH, CH), jnp.int32),
        pltpu.VMEM((CH, D), jnp.float32),
        pltpu.VMEM((CH, D), jnp.float32),
        pltpu.SemaphoreType.DMA,
        pltpu.SemaphoreType.DMA,
        pltpu.SemaphoreType.DMA,
    ],
)
def _spmm_kernel(v_hbm, row2_hbm, col3_hbm, zeros_hbm, z_out,
                 acc, rowidx, colidx, stA, stB, gsA, gsB, zsem):
    c = lax.axis_index("c")
    s = lax.axis_index("s")
    wid = c * NS + s
    @pl.when(c == 0)
    def _():
        pltpu.async_copy(v_hbm.at[pl.ds(s * RPT, RPT)],
                         acc.at[pl.ds(s * RPT, RPT)], zsem)

    @pl.when(c == 1)
    def _():
        pltpu.async_copy(zeros_hbm.at[pl.ds(s * RPT, RPT)],
                         acc.at[pl.ds(s * RPT, RPT)], zsem)

    pltpu.sync_copy(row2_hbm.at[wid], rowidx)
    pltpu.sync_copy(col3_hbm.at[wid], colidx)
    pltpu.make_async_copy(zeros_hbm.at[pl.ds(s * RPT, RPT)],
                          acc.at[pl.ds(s * RPT, RPT)], zsem).wait()
    plsc.subcore_barrier()

    def gstart(j, st, sem):
        pltpu.async_copy(v_hbm.at[rowidx.at[pl.ds(j * CH, CH)]], st, sem)

    def gwait(st, sem):
        pltpu.make_async_copy(v_hbm.at[pl.ds(0, CH)], st, sem).wait()

    gstart(0, stA, gsA)

    def body(j2, carry):
        j = 2 * j2
        gstart(j + 1, stB, gsB)
        gwait(stA, gsA)
        pltpu.sync_copy(stA, acc.at[colidx.at[j]], add=True)
        gstart(j + 2, stA, gsA)
        gwait(stB, gsB)
        pltpu.sync_copy(stB, acc.at[colidx.at[j + 1]], add=True)
        return carry

    lax.fori_loop(0, (NCH - 1) // 2, body, 0)
    gwait(stA, gsA)
    pltpu.sync_copy(stA, acc.at[colidx.at[NCH - 1]], add=True)
    plsc.subcore_barrier()
    pltpu.sync_copy(acc.at[pl.ds(s * RPT, RPT)], z_out.at[c, pl.ds(s * RPT, RPT)])


BN = 10240
G = NPAD // BN


def _proj_body(x_ref, wt_ref, b_ref, rw_ref, out0_ref, roots_ref):
    x = x_ref[...]
    out0_ref[...] = jnp.dot(x, wt_ref[...], preferred_element_type=jnp.float32) + b_ref[...]
    roots_ref[...] = jnp.dot(x, rw_ref[...], preferred_element_type=jnp.float32)


def _proj_call(x, initWT, b2d, rw_cat):
    return pl.pallas_call(
        _proj_body,
        grid=(G,),
        in_specs=[
            pl.BlockSpec((BN, D), lambda i: (i, 0)),
            pl.BlockSpec((D, D), lambda i: (0, 0)),
            pl.BlockSpec((1, D), lambda i: (0, 0)),
            pl.BlockSpec((D, K * D), lambda i: (0, 0)),
        ],
        out_specs=[
            pl.BlockSpec((BN, D), lambda i: (i, 0)),
            pl.BlockSpec((BN, K * D), lambda i: (i, 0)),
        ],
        out_shape=[
            jax.ShapeDtypeStruct((NPAD, D), jnp.float32),
            jax.ShapeDtypeStruct((NPAD, K * D), jnp.float32),
        ],
    )(x, initWT, b2d, rw_cat)


def _scale_body(out0_ref, d0_ref, d1_ref, v_ref, dis_ref):
    deg = 1.0 + d0_ref[...][:, 0:1] + d1_ref[...][:, 0:1]
    dis = lax.rsqrt(deg)
    v_ref[...] = out0_ref[...] * dis
    dis_ref[...] = jnp.broadcast_to(dis, (BN, 8))


def _scale_call(out0, d0, d1):
    return pl.pallas_call(
        _scale_body,
        grid=(G,),
        in_specs=[
            pl.BlockSpec((BN, D), lambda i: (i, 0)),
            pl.BlockSpec((BN, D), lambda i: (i, 0)),
            pl.BlockSpec((BN, D), lambda i: (i, 0)),
        ],
        out_specs=[
            pl.BlockSpec((BN, D), lambda i: (i, 0)),
            pl.BlockSpec((BN, 8), lambda i: (i, 0)),
        ],
        out_shape=[
            jax.ShapeDtypeStruct((NPAD, D), jnp.float32),
            jax.ShapeDtypeStruct((NPAD, 8), jnp.float32),
        ],
    )(out0, d0, d1)


def _make_step_body(has_bias, last):
    def body(*refs):
        if has_bias:
            z_ref, dis_ref, w_ref, roots_ref, bias_ref, o_ref = refs
        else:
            z_ref, dis_ref, w_ref, roots_ref, o_ref = refs
        z = z_ref[...]
        dis = dis_ref[...][:, 0:1]
        h = (z[0] + z[1]) * dis
        o = jnp.dot(h, w_ref[...], preferred_element_type=jnp.float32) + roots_ref[...]
        if has_bias:
            o = jnp.maximum(o + bias_ref[...], 0.0)
        if last:
            m = jnp.max(o, axis=-1, keepdims=True)
            lse = jnp.log(jnp.sum(jnp.exp(o - m), axis=-1, keepdims=True)) + m
            o_ref[...] = o - lse
        else:
            o_ref[...] = o * dis
    return body


def _step_call(z, dis8, w_k, roots, bias_row, k, last):
    has_bias = bias_row is not None
    in_specs = [
        pl.BlockSpec((NC, BN, D), lambda i: (0, i, 0)),
        pl.BlockSpec((BN, 8), lambda i: (i, 0)),
        pl.BlockSpec((D, D), lambda i: (0, 0)),
        pl.BlockSpec((BN, D), lambda i, k=k: (i, k)),
    ]
    args = [z, dis8, w_k, roots]
    if has_bias:
        in_specs.append(pl.BlockSpec((1, D), lambda i: (0, 0)))
        args.append(bias_row)
    return pl.pallas_call(
        _make_step_body(has_bias, last),
        grid=(G,),
        in_specs=in_specs,
        out_specs=pl.BlockSpec((BN, D), lambda i: (i, 0)),
        out_shape=jax.ShapeDtypeStruct((NPAD, D), jnp.float32),
    )(*args)


def kernel(x, edge_index, weight, root_weight, init_W, init_b, bias):
    row = edge_index[0]
    col = edge_index[1]
    row2 = row.reshape(NW, EPW)
    col3 = col.reshape(NW, NCH, CH)
    xp = jnp.pad(x, ((0, NPAD - N), (0, 0)))
    zeros128 = jnp.zeros((NPAD, D), jnp.float32)
    ones128 = jnp.ones((CH, D), jnp.float32)

    dcols = _deg_kernel(col3, zeros128, ones128)

    initWT = init_W.T
    rw_cat = jnp.concatenate([root_weight[0], root_weight[1], root_weight[2]], axis=1)
    b2d = init_b.reshape(1, D)
    out0, roots = _proj_call(xp, initWT, b2d, rw_cat)
    v, dis8 = _scale_call(out0, dcols[0], dcols[1])

    for t in range(T):
        for k in range(K):
            z = _spmm_kernel(v, row2, col3, zeros128)
            has_bias = k == K - 1
            last = t == T - 1 and k == K - 1
            bias_row = bias[t, K - 1].reshape(1, D) if has_bias else None
            v = _step_call(z, dis8, weight[k], roots, bias_row, k, last)
    return v[:N]


# speedup vs baseline: 1.0075x; 1.0075x over previous
import functools

import jax
import jax.numpy as jnp
from jax import lax
from jax.experimental import pallas as pl
from jax.experimental.pallas import tpu as pltpu
from jax.experimental.pallas import tpu_sc as plsc

N = 10000
NPAD = 10240
E = 320000
D = 128
DH = D // 2
K = 3
T = 4

NC, NS = 2, 16
NW = NC * NS
CH = 80
EPW = E // NW
NCH = EPW // CH
LPE = EPW // 16
RPT = NPAD // NS

_mesh = plsc.VectorSubcoreMesh(core_axis_name="c", subcore_axis_name="s")


NSEM = 5


@functools.partial(
    pl.kernel,
    out_type=jax.ShapeDtypeStruct((NC, NPAD, D), jnp.float32),
    mesh=_mesh,
    scratch_types=[
        pltpu.VMEM_SHARED((NPAD, D), jnp.float32),
        pltpu.VMEM((NCH, CH), jnp.int32),
        pltpu.VMEM((CH, D), jnp.float32),
        [pltpu.SemaphoreType.DMA for _ in range(NSEM)],
        pltpu.SemaphoreType.DMA,
    ],
)
def _deg_kernel(col3_hbm, zeros_hbm, ones_hbm, d_out, acc, colidx, ones_v,
                ssems, zsem):
    c = lax.axis_index("c")
    s = lax.axis_index("s")
    wid = c * NS + s
    pltpu.async_copy(zeros_hbm.at[pl.ds(s * RPT, RPT)],
                     acc.at[pl.ds(s * RPT, RPT)], zsem)
    pltpu.sync_copy(col3_hbm.at[wid], colidx)
    pltpu.sync_copy(ones_hbm, ones_v)
    pltpu.make_async_copy(zeros_hbm.at[pl.ds(s * RPT, RPT)],
                          acc.at[pl.ds(s * RPT, RPT)], zsem).wait()
    plsc.subcore_barrier()

    def sstart(j, i):
        pltpu.async_copy(ones_v, acc.at[colidx.at[j]], ssems[i], add=True)

    def swait(i):
        pltpu.make_async_copy(zeros_hbm.at[pl.ds(0, CH)], ones_v,
                              ssems[i]).wait()

    for i in range(NSEM):
        sstart(i, i)

    def body(m, carry):
        for i in range(NSEM):
            swait(i)
            sstart(m * NSEM + i, i)
        return carry

    lax.fori_loop(1, NCH // NSEM, body, 0)
    for i in range(NSEM):
        swait(i)
    plsc.subcore_barrier()
    pltpu.sync_copy(acc.at[pl.ds(s * RPT, RPT)], d_out.at[c, pl.ds(s * RPT, RPT)])


@functools.partial(
    pl.kernel,
    out_type=jax.ShapeDtypeStruct((NC, NPAD, D), jnp.float32),
    mesh=_mesh,
    scratch_types=[
        pltpu.VMEM_SHARED((NPAD, D), jnp.float32),
        pltpu.VMEM((EPW,), jnp.int32),
        pltpu.VMEM((NCH, CH), jnp.int32),
        pltpu.VMEM((CH, D), jnp.float32),
        pltpu.VMEM((CH, D), jnp.float32),
        pltpu.SemaphoreType.DMA,
        pltpu.SemaphoreType.DMA,
        pltpu.SemaphoreType.DMA,
    ],
)
def _spmm_kernel(v_hbm, row2_hbm, col3_hbm, zeros_hbm, z_out,
                 acc, rowidx, colidx, stA, stB, gsA, gsB, zsem):
    c = lax.axis_index("c")
    s = lax.axis_index("s")
    wid = c * NS + s
    @pl.when(c == 0)
    def _():
        pltpu.async_copy(v_hbm.at[pl.ds(s * RPT, RPT)],
                         acc.at[pl.ds(s * RPT, RPT)], zsem)

    @pl.when(c == 1)
    def _():
        pltpu.async_copy(zeros_hbm.at[pl.ds(s * RPT, RPT)],
                         acc.at[pl.ds(s * RPT, RPT)], zsem)

    pltpu.sync_copy(row2_hbm.at[wid], rowidx)
    pltpu.sync_copy(col3_hbm.at[wid], colidx)
    pltpu.make_async_copy(zeros_hbm.at[pl.ds(s * RPT, RPT)],
                          acc.at[pl.ds(s * RPT, RPT)], zsem).wait()
    plsc.subcore_barrier()

    def gstart(j, st, sem):
        pltpu.async_copy(v_hbm.at[rowidx.at[pl.ds(j * CH, CH)]], st, sem)

    def gwait(st, sem):
        pltpu.make_async_copy(v_hbm.at[pl.ds(0, CH)], st, sem).wait()

    gstart(0, stA, gsA)

    def body(j2, carry):
        j = 2 * j2
        gstart(j + 1, stB, gsB)
        gwait(stA, gsA)
        pltpu.sync_copy(stA, acc.at[colidx.at[j]], add=True)
        gstart(j + 2, stA, gsA)
        gwait(stB, gsB)
        pltpu.sync_copy(stB, acc.at[colidx.at[j + 1]], add=True)
        return carry

    lax.fori_loop(0, (NCH - 1) // 2, body, 0)
    gwait(stA, gsA)
    pltpu.sync_copy(stA, acc.at[colidx.at[NCH - 1]], add=True)
    plsc.subcore_barrier()
    pltpu.sync_copy(acc.at[pl.ds(s * RPT, RPT)], z_out.at[c, pl.ds(s * RPT, RPT)])


BN = 5120
G = NPAD // BN


def _proj_body(x_ref, wt_ref, b_ref, rw_ref, out0_ref, roots_ref):
    x = x_ref[...]
    out0_ref[...] = jnp.dot(x, wt_ref[...], preferred_element_type=jnp.float32) + b_ref[...]
    roots_ref[...] = jnp.dot(x, rw_ref[...], preferred_element_type=jnp.float32)


def _proj_call(x, initWT, b2d, rw_cat):
    return pl.pallas_call(
        _proj_body,
        grid=(G,),
        in_specs=[
            pl.BlockSpec((BN, D), lambda i: (i, 0)),
            pl.BlockSpec((D, D), lambda i: (0, 0)),
            pl.BlockSpec((1, D), lambda i: (0, 0)),
            pl.BlockSpec((D, K * D), lambda i: (0, 0)),
        ],
        out_specs=[
            pl.BlockSpec((BN, D), lambda i: (i, 0)),
            pl.BlockSpec((BN, K * D), lambda i: (i, 0)),
        ],
        out_shape=[
            jax.ShapeDtypeStruct((NPAD, D), jnp.float32),
            jax.ShapeDtypeStruct((NPAD, K * D), jnp.float32),
        ],
    )(x, initWT, b2d, rw_cat)


def _scale_body(out0_ref, d0_ref, d1_ref, v_ref, dis_ref):
    deg = 1.0 + d0_ref[...][:, 0:1] + d1_ref[...][:, 0:1]
    dis = lax.rsqrt(deg)
    v_ref[...] = out0_ref[...] * dis
    dis_ref[...] = jnp.broadcast_to(dis, (BN, 8))


def _scale_call(out0, d0, d1):
    return pl.pallas_call(
        _scale_body,
        grid=(G,),
        in_specs=[
            pl.BlockSpec((BN, D), lambda i: (i, 0)),
            pl.BlockSpec((BN, D), lambda i: (i, 0)),
            pl.BlockSpec((BN, D), lambda i: (i, 0)),
        ],
        out_specs=[
            pl.BlockSpec((BN, D), lambda i: (i, 0)),
            pl.BlockSpec((BN, 8), lambda i: (i, 0)),
        ],
        out_shape=[
            jax.ShapeDtypeStruct((NPAD, D), jnp.float32),
            jax.ShapeDtypeStruct((NPAD, 8), jnp.float32),
        ],
    )(out0, d0, d1)


def _make_step_body(has_bias, last):
    def body(*refs):
        if has_bias:
            z_ref, dis_ref, w_ref, roots_ref, bias_ref, o_ref = refs
        else:
            z_ref, dis_ref, w_ref, roots_ref, o_ref = refs
        z = z_ref[...]
        dis = dis_ref[...][:, 0:1]
        h = (z[0] + z[1]) * dis
        o = jnp.dot(h, w_ref[...], preferred_element_type=jnp.float32) + roots_ref[...]
        if has_bias:
            o = jnp.maximum(o + bias_ref[...], 0.0)
        if last:
            m = jnp.max(o, axis=-1, keepdims=True)
            lse = jnp.log(jnp.sum(jnp.exp(o - m), axis=-1, keepdims=True)) + m
            o_ref[...] = o - lse
        else:
            o_ref[...] = o * dis
    return body


def _step_call(z, dis8, w_k, roots, bias_row, k, last):
    has_bias = bias_row is not None
    in_specs = [
        pl.BlockSpec((NC, BN, D), lambda i: (0, i, 0)),
        pl.BlockSpec((BN, 8), lambda i: (i, 0)),
        pl.BlockSpec((D, D), lambda i: (0, 0)),
        pl.BlockSpec((BN, D), lambda i, k=k: (i, k)),
    ]
    args = [z, dis8, w_k, roots]
    if has_bias:
        in_specs.append(pl.BlockSpec((1, D), lambda i: (0, 0)))
        args.append(bias_row)
    return pl.pallas_call(
        _make_step_body(has_bias, last),
        grid=(G,),
        in_specs=in_specs,
        out_specs=pl.BlockSpec((BN, D), lambda i: (i, 0)),
        out_shape=jax.ShapeDtypeStruct((NPAD, D), jnp.float32),
    )(*args)


def kernel(x, edge_index, weight, root_weight, init_W, init_b, bias):
    row = edge_index[0]
    col = edge_index[1]
    row2 = row.reshape(NW, EPW)
    col3 = col.reshape(NW, NCH, CH)
    xp = jnp.pad(x, ((0, NPAD - N), (0, 0)))
    zeros128 = jnp.zeros((NPAD, D), jnp.float32)
    ones128 = jnp.ones((CH, D), jnp.float32)

    dcols = _deg_kernel(col3, zeros128, ones128)

    initWT = init_W.T
    rw_cat = jnp.concatenate([root_weight[0], root_weight[1], root_weight[2]], axis=1)
    b2d = init_b.reshape(1, D)
    out0, roots = _proj_call(xp, initWT, b2d, rw_cat)
    v, dis8 = _scale_call(out0, dcols[0], dcols[1])

    for t in range(T):
        for k in range(K):
            z = _spmm_kernel(v, row2, col3, zeros128)
            has_bias = k == K - 1
            last = t == T - 1 and k == K - 1
            bias_row = bias[t, K - 1].reshape(1, D) if has_bias else None
            v = _step_call(z, dis8, weight[k], roots, bias_row, k, last)
    return v[:N]
